# baseline (speedup 1.0000x reference)
import functools

import jax
import jax.numpy as jnp
from jax import lax
from jax.experimental import pallas as pl
from jax.experimental.pallas import tpu as pltpu
from jax.experimental.pallas import tpu_sc as plsc

BATCH = 16384
DIM = 32
NSLOT = 8
PACK = 4
WIDE = PACK * DIM
NC, NS = 2, 16
NW = NC * NS
BPW = BATCH // NW
CHUNK = 64
NCHUNK = BPW // CHUNK
OUT_D = NSLOT * DIM
NBUF = 2
LANES = 16

_mesh = plsc.VectorSubcoreMesh(core_axis_name="c", subcore_axis_name="s")


def _repack_table(table):
    return jnp.concatenate([table[a::PACK, :] for a in range(PACK)], axis=1)


@functools.partial(
    pl.kernel,
    out_type=jax.ShapeDtypeStruct((BATCH, OUT_D), jnp.float32),
    mesh=_mesh,
    scratch_types=[
        pltpu.VMEM((NSLOT * BPW,), jnp.int32),
        pltpu.VMEM((NBUF, NSLOT * CHUNK), jnp.int32),
        pltpu.VMEM((NSLOT * CHUNK, WIDE), jnp.float32),
        pltpu.VMEM((NBUF, CHUNK, OUT_D), jnp.float32),
        pltpu.SemaphoreType.DMA,
        pltpu.SemaphoreType.DMA,
        pltpu.SemaphoreType.DMA,
        pltpu.SemaphoreType.DMA,
    ],
    compiler_params=pltpu.CompilerParams(needs_layout_passes=False),
)
def _gather_concat(q_hbm, rem_hbm, paper_hbm, pfield_hbm, author_hbm,
                   year_hbm, oa_hbm, out_hbm, q_v, rem_v, wide_v, asm_v,
                   gsem, rsem, wsem0, wsem1):
    wid = lax.axis_index("s") * NC + lax.axis_index("c")
    base = wid * BPW
    tables = (paper_hbm, pfield_hbm, pfield_hbm, author_hbm, author_hbm,
              author_hbm, year_hbm, oa_hbm)
    wsems = (wsem0, wsem1)
    pltpu.sync_copy(q_hbm.at[pl.ds(wid * NSLOT * BPW, NSLOT * BPW)], q_v)

    def issue_rem(c):
        return pltpu.async_copy(
            rem_hbm.at[pl.ds((wid * NCHUNK + c) * NSLOT * CHUNK,
                             NSLOT * CHUNK)],
            rem_v.at[c % NBUF], rsem)

    def issue_gathers(c):
        return [
            pltpu.async_copy(
                tab.at[q_v.at[pl.ds(s * BPW + c * CHUNK, CHUNK)]],
                wide_v.at[pl.ds(s * CHUNK, CHUNK)], gsem)
            for s, tab in enumerate(tables)
        ]

    def repack(buf):
        def grp(t, _):
            g = t // NSLOT
            s = t % NSLOT
            rv = rem_v[buf, pl.ds(s * CHUNK + g * LANES, LANES)] * DIM
            for j in range(LANES):
                r = g * LANES + j
                src = s * CHUNK + r
                off = rv[j]
                for k in range(DIM // LANES):
                    asm_v[buf, r, pl.ds(s * DIM + k * LANES, LANES)] = (
                        wide_v[src, pl.ds(off + k * LANES, LANES)])
            return 0
        lax.fori_loop(0, (CHUNK // LANES) * NSLOT, grp, 0)

    writes = [None] * NBUF
    rems = [issue_rem(0), None]
    gathers = issue_gathers(0)
    for c in range(NCHUNK):
        buf = c % NBUF
        if c + 1 < NCHUNK:
            rems[(c + 1) % NBUF] = issue_rem(c + 1)
        for g in gathers:
            g.wait()
        rems[buf].wait()
        if writes[buf] is not None:
            writes[buf].wait()
        repack(buf)
        if c + 1 < NCHUNK:
            gathers = issue_gathers(c + 1)
        writes[buf] = pltpu.async_copy(
            asm_v.at[buf], out_hbm.at[pl.ds(base + c * CHUNK, CHUNK)],
            wsems[buf])
    for w in writes:
        if w is not None:
            w.wait()


def kernel(paperId, fieldsOfStudy_0, fieldsOfStudy_1, authors_0, authors_1,
           authors_2, year, isOpenAccess, paper_table, pfield_table,
           author_table, year_table, oa_table):
    idx = jnp.stack([paperId, fieldsOfStudy_0, fieldsOfStudy_1, authors_0,
                     authors_1, authors_2, year, isOpenAccess])
    idx = idx.astype(jnp.int32).reshape(NSLOT, NW, NCHUNK, CHUNK)
    q = (idx // PACK).transpose(1, 0, 2, 3).reshape(-1)
    rem = (idx % PACK).transpose(1, 2, 0, 3).reshape(-1)
    oa_pad = jnp.pad(oa_table, ((0, 1), (0, 0)))
    return _gather_concat(
        q, rem,
        _repack_table(paper_table), _repack_table(pfield_table),
        _repack_table(author_table), _repack_table(year_table),
        _repack_table(oa_pad))


# speedup vs baseline: 8.6943x; 8.6943x over previous
import functools

import jax
import jax.numpy as jnp
from jax import lax
from jax.experimental import pallas as pl
from jax.experimental.pallas import tpu as pltpu
from jax.experimental.pallas import tpu_sc as plsc

BATCH = 16384
DIM = 32
NSLOT = 8
NC, NS = 2, 16
NW = NC * NS
BPW = BATCH // NW
CHUNK = 64
NCHUNK = BPW // CHUNK
OUT_D = NSLOT * DIM
NBUF = 4

_mesh = plsc.VectorSubcoreMesh(core_axis_name="c", subcore_axis_name="s")


@functools.partial(
    pl.kernel,
    out_type=jax.ShapeDtypeStruct((BATCH, OUT_D), jnp.float32),
    mesh=_mesh,
    scratch_types=[
        pltpu.VMEM((NSLOT * BPW,), jnp.int32),
        pltpu.VMEM((NBUF, NSLOT, CHUNK, DIM), jnp.float32),
        pltpu.SemaphoreType.DMA,
        pltpu.SemaphoreType.DMA,
        pltpu.SemaphoreType.DMA,
    ],
    compiler_params=pltpu.CompilerParams(use_tc_tiling_on_sc=False),
)
def _gather_concat(idx_hbm, paper_hbm, pfield_hbm, author_hbm, year_hbm,
                   oa_hbm, out_hbm, idx_v, slot_v, gsem, wsem0, wsem1):
    wid = lax.axis_index("s") * NC + lax.axis_index("c")
    base = wid * BPW
    tables = (paper_hbm, pfield_hbm, pfield_hbm, author_hbm, author_hbm,
              author_hbm, year_hbm, oa_hbm)
    wsems = (wsem0, wsem1)
    pltpu.sync_copy(idx_hbm.at[pl.ds(wid * NSLOT * BPW, NSLOT * BPW)], idx_v)

    def issue_gathers(c):
        buf = c % NBUF
        return [
            pltpu.async_copy(
                tab.at[idx_v.at[pl.ds(s * BPW + c * CHUNK, CHUNK)]],
                slot_v.at[buf, s], gsem)
            for s, tab in enumerate(tables)
        ]

    def issue_writes(c):
        buf = c % NBUF
        rb = base + c * CHUNK
        return [
            pltpu.async_copy(
                slot_v.at[buf, s],
                out_hbm.at[pl.ds(rb, CHUNK), pl.ds(s * DIM, DIM)],
                wsems[c % 2])
            for s in range(NSLOT)
        ]

    gathers = [None] * NCHUNK
    writes = [None] * NCHUNK
    for b in range(NBUF - 1):
        gathers[b] = issue_gathers(b)
    for c in range(NCHUNK):
        n = c + NBUF - 1
        if n < NCHUNK:
            if c >= 1 and writes[c - 1] is not None:
                for w in writes[c - 1]:
                    w.wait()
            gathers[n] = issue_gathers(n)
        for g in gathers[c]:
            g.wait()
        writes[c] = issue_writes(c)
    for c in range(NCHUNK):
        if writes[c] is not None and c >= NCHUNK - NBUF:
            for w in writes[c]:
                w.wait()


def kernel(paperId, fieldsOfStudy_0, fieldsOfStudy_1, authors_0, authors_1,
           authors_2, year, isOpenAccess, paper_table, pfield_table,
           author_table, year_table, oa_table):
    idx = jnp.stack([paperId, fieldsOfStudy_0, fieldsOfStudy_1, authors_0,
                     authors_1, authors_2, year, isOpenAccess])
    idx = (idx.astype(jnp.int32)
              .reshape(NSLOT, NW, BPW)
              .transpose(1, 0, 2)
              .reshape(-1))
    one = (paperId[0] * 0 + 1).astype(jnp.float32)
    return _gather_concat(idx, paper_table * one, pfield_table,
                          author_table * one, year_table, oa_table)
